# Initial kernel scaffold; baseline (speedup 1.0000x reference)
import jax
import jax.numpy as jnp
from jax.experimental import pallas as pl


def kernel(jets, W_emb, b_emb, W_mp0, b_mp0, W_mp1, b_mp1, W_mp2, b_mp2, W_r1, b_r1, W_r2, b_r2):
    raise NotImplementedError("write your pallas kernel here")


# speedup vs baseline: 1.2111x; 1.2111x over previous
import functools

import jax
import jax.numpy as jnp
import numpy as np
from jax.experimental import pallas as pl
from jax.experimental.pallas import tpu as pltpu

_B, _N, _F_IN, _HID, _ITERS = 128, 200, 8, 256, 3
_SCALE = 1.0 / np.sqrt(_HID)


def _mm(a, b):
    return jax.lax.dot_general(
        a, b, (((1,), (0,)), ((), ())), preferred_element_type=jnp.float32
    )


def _mpnn_kernel(jets_ref, w_emb_ref, b_emb_ref,
                 w_mp0_ref, b_mp0_ref, w_mp1_ref, b_mp1_ref,
                 w_mp2_ref, b_mp2_ref,
                 w_r1_ref, b_r1_ref, w_r2_ref, b_r2_ref,
                 out_ref, a_ref):
    x = jets_ref[0]
    h = jnp.tanh(_mm(x, w_emb_ref[...]) + b_emb_ref[...])

    a = None
    for w_ref, b_ref in ((w_mp0_ref, b_mp0_ref),
                         (w_mp1_ref, b_mp1_ref),
                         (w_mp2_ref, b_mp2_ref)):
        logits = jax.lax.dot_general(
            h, h, (((1,), (1,)), ((), ())),
            preferred_element_type=jnp.float32) * _SCALE
        m = jnp.max(logits, axis=-1, keepdims=True)
        p = jnp.exp(logits - m)
        a = p / jnp.sum(p, axis=-1, keepdims=True)
        msg = _mm(a, h)
        w = w_ref[...]
        upd = _mm(h, w[:_HID]) + _mm(msg, w[_HID:]) + b_ref[...]
        h = jnp.tanh(upd)

    r = jnp.tanh(_mm(h, w_r1_ref[...]) + b_r1_ref[...])
    r2 = _mm(r, w_r2_ref[...])
    out_ref[0] = jnp.sum(r2, axis=0, keepdims=True) + _N * b_r2_ref[...]
    a_ref[0] = a


def kernel(jets, W_emb, b_emb, W_mp0, b_mp0, W_mp1, b_mp1, W_mp2, b_mp2,
           W_r1, b_r1, W_r2, b_r2):
    B, N, F_IN = jets.shape
    HID = W_emb.shape[1]

    def rep(shape):
        return pl.BlockSpec(shape, lambda b: (0,) * len(shape))

    b_emb2 = b_emb.reshape(1, HID)
    b_mp0_2 = b_mp0.reshape(1, HID)
    b_mp1_2 = b_mp1.reshape(1, HID)
    b_mp2_2 = b_mp2.reshape(1, HID)
    b_r1_2 = b_r1.reshape(1, HID)
    b_r2_2 = b_r2.reshape(1, HID)

    out, a = pl.pallas_call(
        _mpnn_kernel,
        grid=(B,),
        in_specs=[
            pl.BlockSpec((1, N, F_IN), lambda b: (b, 0, 0)),
            rep((F_IN, HID)), rep((1, HID)),
            rep((2 * HID, HID)), rep((1, HID)),
            rep((2 * HID, HID)), rep((1, HID)),
            rep((2 * HID, HID)), rep((1, HID)),
            rep((HID, HID)), rep((1, HID)),
            rep((HID, HID)), rep((1, HID)),
        ],
        out_specs=[
            pl.BlockSpec((1, 1, HID), lambda b: (b, 0, 0)),
            pl.BlockSpec((1, N, N), lambda b: (b, 0, 0)),
        ],
        out_shape=[
            jax.ShapeDtypeStruct((B, 1, HID), jnp.float32),
            jax.ShapeDtypeStruct((B, N, N), jnp.float32),
        ],
        compiler_params=pltpu.CompilerParams(
            dimension_semantics=("parallel",),
        ),
    )(jets, W_emb, b_emb2, W_mp0, b_mp0_2, W_mp1, b_mp1_2, W_mp2, b_mp2_2,
      W_r1, b_r1_2, W_r2, b_r2_2)
    return (out.reshape(B, HID), a)
